# baseline (speedup 1.0000x reference)
import functools

import jax
import jax.numpy as jnp
from jax.experimental import pallas as pl

_IBLK = 64


def _mm(a, w):
    return jax.lax.dot_general(a, w, (((a.ndim - 1,), (0,)), ((), ())),
                               preferred_element_type=jnp.float32)


def _mlp(h, params, last_linear=True):
    n = len(params)
    for k, (w, b) in enumerate(params):
        h = _mm(h, w) + b
        if k < n - 1 or not last_linear:
            h = jnp.maximum(h, 0.0)
    return h


def _gnn_kernel(x_ref, p_ref, xi_ref, pi_ref, *refs,
                n_enc, n_postgnn, n_local, n_post):
    total_pairs = n_enc + 3 + n_postgnn + n_local + n_post
    flat = refs[:2 * total_pairs]
    o_ref = refs[2 * total_pairs]
    vals = [r[...] for r in flat]
    pairs = [(vals[2 * k], vals[2 * k + 1]) for k in range(total_pairs)]
    enc_p = pairs[:n_enc]
    gnn_p = pairs[n_enc:n_enc + 3]
    pg_p = pairs[n_enc + 3:n_enc + 3 + n_postgnn]
    loc_p = pairs[n_enc + 3 + n_postgnn:n_enc + 3 + n_postgnn + n_local]
    post_p = pairs[n_enc + 3 + n_postgnn + n_local:]

    x = x_ref[0]
    p = p_ref[0]
    x_i = xi_ref[0]
    p_i = pi_ref[0]
    N = x.shape[0]
    I = x_i.shape[0]
    ib = pl.program_id(1)
    i0 = ib * I

    enc = _mlp(x, enc_p)
    (w1, b1), (w2, b2), (w3, b3) = gnn_p
    u = _mm(enc, w1)
    u_i = _mm(_mlp(x_i, enc_p), w1)

    d2 = jnp.sum((p_i[:, None, :] - p[None, :, :]) ** 2, axis=-1)
    jg = jax.lax.broadcasted_iota(jnp.int32, (I, N), 1)
    ig = jax.lax.broadcasted_iota(jnp.int32, (I, N), 0) + i0
    mask = (d2 < 1.0) & (jg != ig)

    h1 = jnp.maximum(u[None, :, :] - u_i[:, None, :] + b1[None, None, :], 0.0)
    h2 = jnp.maximum(_mm(h1.reshape(I * N, 64), w2) + b2, 0.0)
    h2 = h2.reshape(I, N, 64)

    mf = mask.astype(jnp.float32)
    agg = jnp.sum(h2 * mf[:, :, None], axis=1)
    deg = jnp.sum(mf, axis=1, keepdims=True)
    gnn_out = _mm(agg, w3) + deg * b3

    post_gnn = _mlp(gnn_out, pg_p)
    local = _mlp(x_i, loc_p)
    o_ref[0] = _mlp(post_gnn + local, post_p)


def kernel(x, p, comm_radius, enc_params, gnn_params, post_gnn_params,
           local_params, post_params):
    B, N, _ = x.shape
    I = _IBLK
    p_scaled = p / jnp.asarray(comm_radius, jnp.float32)

    weight_arrays = []
    for group in (enc_params, gnn_params, post_gnn_params, local_params,
                  post_params):
        for w, b in group:
            weight_arrays.append(w)
            weight_arrays.append(b.reshape(1, -1))

    grid = (B, N // I)
    in_specs = [
        pl.BlockSpec((1, N, x.shape[2]), lambda b, i: (b, 0, 0)),
        pl.BlockSpec((1, N, p.shape[2]), lambda b, i: (b, 0, 0)),
        pl.BlockSpec((1, I, x.shape[2]), lambda b, i: (b, i, 0)),
        pl.BlockSpec((1, I, p.shape[2]), lambda b, i: (b, i, 0)),
    ] + [pl.BlockSpec(w.shape, lambda b, i: (0, 0)) for w in weight_arrays]

    out = pl.pallas_call(
        functools.partial(_gnn_kernel, n_enc=len(enc_params),
                          n_postgnn=len(post_gnn_params),
                          n_local=len(local_params), n_post=len(post_params)),
        grid=grid,
        in_specs=in_specs,
        out_specs=pl.BlockSpec((1, I, 32), lambda b, i: (b, i, 0)),
        out_shape=jax.ShapeDtypeStruct((B, N, 32), jnp.float32),
    )(x, p_scaled, x, p_scaled, *weight_arrays)
    return out


# speedup vs baseline: 1.7234x; 1.7234x over previous
import functools

import jax
import jax.numpy as jnp
from jax.experimental import pallas as pl

_IBLK = 64


def _mm(a, w):
    return jax.lax.dot_general(a, w, (((a.ndim - 1,), (0,)), ((), ())),
                               preferred_element_type=jnp.float32)


def _mlp(h, params, last_linear=True):
    n = len(params)
    for k, (w, b) in enumerate(params):
        h = _mm(h, w) + b
        if k < n - 1 or not last_linear:
            h = jnp.maximum(h, 0.0)
    return h


def _gnn_kernel(x_ref, pt_ref, xi_ref, pi_ref, *refs,
                n_enc, n_postgnn, n_local, n_post):
    total_pairs = n_enc + 3 + n_postgnn + n_local + n_post
    flat = refs[:2 * total_pairs]
    o_ref = refs[2 * total_pairs]
    vals = [r[...] for r in flat]
    pairs = [(vals[2 * k], vals[2 * k + 1]) for k in range(total_pairs)]
    enc_p = pairs[:n_enc]
    gnn_p = pairs[n_enc:n_enc + 3]
    pg_p = pairs[n_enc + 3:n_enc + 3 + n_postgnn]
    loc_p = pairs[n_enc + 3 + n_postgnn:n_enc + 3 + n_postgnn + n_local]
    post_p = pairs[n_enc + 3 + n_postgnn + n_local:]

    x = x_ref[0]
    pt = pt_ref[0]
    x_i = xi_ref[0]
    p_i = pi_ref[0]
    N = x.shape[0]
    I = x_i.shape[0]

    enc = _mlp(x, enc_p)
    (w1, b1), (w2, b2), (w3, b3) = gnn_p
    u = _mm(enc, w1)
    u_i = _mm(_mlp(x_i, enc_p), w1)

    dx = p_i[:, 0:1] - pt[0:1, :]
    dy = p_i[:, 1:2] - pt[1:2, :]
    d2 = dx * dx + dy * dy
    mask = d2 < 1.0

    h1 = jnp.maximum((u[None, :, :] - u_i[:, None, :]) + b1, 0.0)
    h2 = jnp.maximum(_mm(h1.reshape(I * N, 64), w2) + b2, 0.0)
    h2 = h2.reshape(I, N, 64)

    mf = jnp.where(mask, 1.0, 0.0)
    agg = jnp.sum(h2 * mf[:, :, None], axis=1)
    deg = jnp.sum(mf, axis=1, keepdims=True) - 1.0
    s2 = jnp.maximum(_mm(jnp.maximum(b1, 0.0), w2) + b2, 0.0)
    gnn_out = _mm(agg - s2, w3) + deg * b3

    post_gnn = _mlp(gnn_out, pg_p)
    local = _mlp(x_i, loc_p)
    o_ref[0] = _mlp(post_gnn + local, post_p)


def kernel(x, p, comm_radius, enc_params, gnn_params, post_gnn_params,
           local_params, post_params):
    B, N, _ = x.shape
    I = _IBLK
    p_scaled = p / jnp.asarray(comm_radius, jnp.float32)
    pt = jnp.swapaxes(p_scaled, 1, 2)

    weight_arrays = []
    for group in (enc_params, gnn_params, post_gnn_params, local_params,
                  post_params):
        for w, b in group:
            weight_arrays.append(w)
            weight_arrays.append(b.reshape(1, -1))

    grid = (B, N // I)
    in_specs = [
        pl.BlockSpec((1, N, x.shape[2]), lambda b, i: (b, 0, 0)),
        pl.BlockSpec((1, 2, N), lambda b, i: (b, 0, 0)),
        pl.BlockSpec((1, I, x.shape[2]), lambda b, i: (b, i, 0)),
        pl.BlockSpec((1, I, p.shape[2]), lambda b, i: (b, i, 0)),
    ] + [pl.BlockSpec(w.shape, lambda b, i: (0, 0)) for w in weight_arrays]

    out = pl.pallas_call(
        functools.partial(_gnn_kernel, n_enc=len(enc_params),
                          n_postgnn=len(post_gnn_params),
                          n_local=len(local_params), n_post=len(post_params)),
        grid=grid,
        in_specs=in_specs,
        out_specs=pl.BlockSpec((1, I, 32), lambda b, i: (b, i, 0)),
        out_shape=jax.ShapeDtypeStruct((B, N, 32), jnp.float32),
    )(x, pt, x, p_scaled, *weight_arrays)
    return out
